# baseline (speedup 1.0000x reference)
import functools

import jax
import jax.numpy as jnp
from jax import lax
from jax.experimental import pallas as pl
from jax.experimental.pallas import tpu as pltpu
from jax.experimental.pallas import tpu_sc as plsc

B = 4096
K = 8192
D = 4096
BM = 512
BK = 512
TILE = 2048
TB = TILE // BK
NB = B // BM
NK = K // BK


def _dist_argmin_kernel(z_ref, c_ref, zsq_ref, csq_ref, idx_ref, mind_ref,
                        dotbuf, tile_min, tile_idx, run_v, run_i, run_d):
    k = pl.program_id(1)

    zb = z_ref[...]
    cb = c_ref[...]
    dotbuf[k % 2] = lax.dot_general(
        zb, cb, (((1,), (1,)), ((), ())),
        preferred_element_type=jnp.float32)

    kk = jnp.maximum(k - 1, 0)
    d = (zsq_ref[...] + csq_ref[...]) - 2.0 * dotbuf[(k + 1) % 2]

    lmin = jnp.min(d, axis=1, keepdims=True)
    cols = lax.broadcasted_iota(jnp.int32, (1, BK), 1)
    lidx = jnp.min(jnp.where(d == lmin, cols, K), axis=1,
                   keepdims=True) + kk * BK

    @pl.when(kk % TB == 0)
    def _():
        tile_min[...] = lmin
        tile_idx[...] = lidx

    @pl.when(kk % TB != 0)
    def _():
        better = lmin < tile_min[...]
        tile_min[...] = jnp.where(better, lmin, tile_min[...])
        tile_idx[...] = jnp.where(better, lidx, tile_idx[...])

    @pl.when((k > 0) & (kk % TB == TB - 1))
    def _():
        m = tile_min[...]
        ti = tile_idx[...]

        @pl.when(kk == TB - 1)
        def _():
            run_i[...] = ti
            run_d[...] = m
            run_v[...] = m.astype(jnp.bfloat16).astype(jnp.float32)

        @pl.when(kk > TB - 1)
        def _():
            take = m < run_v[...]
            run_i[...] = jnp.where(take, ti, run_i[...])
            run_d[...] = jnp.where(take, m, run_d[...])
            run_v[...] = jnp.where(take, m, run_v[...]).astype(
                jnp.bfloat16).astype(jnp.float32)

    @pl.when(k == NK)
    def _():
        idx_ref[0] = run_i[...]
        mind_ref[0] = run_d[...]


def _dist_argmin(z_flat, codebook, zsq, csq):
    idx, mind = pl.pallas_call(
        _dist_argmin_kernel,
        grid=(NB, NK + 1),
        in_specs=[
            pl.BlockSpec((BM, D), lambda i, k: (i, 0)),
            pl.BlockSpec((BK, D), lambda i, k: (jnp.minimum(k, NK - 1), 0)),
            pl.BlockSpec((BM, 1), lambda i, k: (i, 0)),
            pl.BlockSpec((1, BK), lambda i, k: (0, jnp.maximum(k - 1, 0))),
        ],
        out_specs=[
            pl.BlockSpec((1, BM, 1), lambda i, k: (i, 0, 0)),
            pl.BlockSpec((1, BM, 1), lambda i, k: (i, 0, 0)),
        ],
        out_shape=[
            jax.ShapeDtypeStruct((NB, BM, 1), jnp.int32),
            jax.ShapeDtypeStruct((NB, BM, 1), jnp.float32),
        ],
        scratch_shapes=[
            pltpu.VMEM((2, BM, BK), jnp.float32),
            pltpu.VMEM((BM, 1), jnp.float32),
            pltpu.VMEM((BM, 1), jnp.int32),
            pltpu.VMEM((BM, 1), jnp.float32),
            pltpu.VMEM((BM, 1), jnp.int32),
            pltpu.VMEM((BM, 1), jnp.float32),
        ],
        compiler_params=pltpu.CompilerParams(
            dimension_semantics=("parallel", "arbitrary")),
    )(z_flat, codebook, zsq, csq)
    return idx.reshape(B), mind.reshape(B)


_SC_INFO = plsc.get_sparse_core_info()
_NC = _SC_INFO.num_cores
_NS = _SC_INFO.num_subcores
_NW = _NC * _NS
_BPW = B // _NW
_CH = 8
_NCH = _BPW // _CH


def _gather_sc_kernel(idx_hbm, table_hbm, out_hbm, idx_v, buf0, buf1,
                      sem0, sem1):
    wid = lax.axis_index("s") * _NC + lax.axis_index("c")
    base = wid * _BPW
    pltpu.sync_copy(idx_hbm.at[pl.ds(base, _BPW)], idx_v)
    bufs = (buf0, buf1)
    sems = (sem0, sem1)
    copies = []
    for t in range(_NCH):
        cp = pltpu.async_copy(
            table_hbm.at[idx_v.at[pl.ds(t * _CH, _CH)]], bufs[t % 2],
            sems[t % 2])
        copies.append(cp)
        if t >= 1:
            copies[t - 1].wait()
            pltpu.sync_copy(bufs[(t - 1) % 2],
                            out_hbm.at[pl.ds(base + (t - 1) * _CH, _CH)])
    copies[-1].wait()
    pltpu.sync_copy(bufs[(_NCH - 1) % 2],
                    out_hbm.at[pl.ds(base + (_NCH - 1) * _CH, _CH)])


@functools.partial(
    pl.kernel,
    mesh=plsc.VectorSubcoreMesh(core_axis_name="c", subcore_axis_name="s"),
    out_type=jax.ShapeDtypeStruct((B, D), jnp.float32),
    scratch_types=[
        pltpu.VMEM((_BPW,), jnp.int32),
        pltpu.VMEM((_CH, D), jnp.float32),
        pltpu.VMEM((_CH, D), jnp.float32),
        pltpu.SemaphoreType.DMA,
        pltpu.SemaphoreType.DMA,
    ],
)
def _gather(idx_hbm, table_hbm, out_hbm, idx_v, buf0, buf1, sem0, sem1):
    _gather_sc_kernel(idx_hbm, table_hbm, out_hbm, idx_v, buf0, buf1,
                      sem0, sem1)


def kernel(z, codebook):
    z_flat = z.reshape(B, -1)
    zsq = jnp.sum(z_flat ** 2, axis=1, keepdims=True)
    csq = jnp.sum(codebook ** 2, axis=1).reshape(1, K)
    indices, mind = _dist_argmin(z_flat, codebook, zsq, csq)
    quantized_flat = _gather(indices, codebook)
    loss = 1.25 * (jnp.sum(mind) / (B * D))
    quantized = quantized_flat.reshape(z.shape)
    return (loss, quantized, indices)


# speedup vs baseline: 1.0782x; 1.0782x over previous
import functools

import jax
import jax.numpy as jnp
from jax import lax
from jax.experimental import pallas as pl
from jax.experimental.pallas import tpu as pltpu
from jax.experimental.pallas import tpu_sc as plsc

B = 4096
K = 8192
D = 4096
BM = 1024
BK = 512
TILE = 2048
TB = TILE // BK
NB = B // BM
NK = K // BK


def _dist_argmin_kernel(z_ref, c_ref, zsq_ref, csq_ref, idx_ref, mind_ref,
                        dotbuf, tile_min, tile_idx, run_v, run_i, run_d):
    k = pl.program_id(1)

    zb = z_ref[...]
    cb = c_ref[...]
    dotbuf[k % 2] = lax.dot_general(
        zb, cb, (((1,), (1,)), ((), ())),
        preferred_element_type=jnp.float32)

    kk = jnp.maximum(k - 1, 0)
    d = (zsq_ref[...] + csq_ref[...]) - 2.0 * dotbuf[(k + 1) % 2]

    lmin = jnp.min(d, axis=1, keepdims=True)
    cols = lax.broadcasted_iota(jnp.int32, (1, BK), 1)
    lidx = jnp.min(jnp.where(d == lmin, cols, K), axis=1,
                   keepdims=True) + kk * BK

    @pl.when(kk % TB == 0)
    def _():
        tile_min[...] = lmin
        tile_idx[...] = lidx

    @pl.when(kk % TB != 0)
    def _():
        better = lmin < tile_min[...]
        tile_min[...] = jnp.where(better, lmin, tile_min[...])
        tile_idx[...] = jnp.where(better, lidx, tile_idx[...])

    @pl.when((k > 0) & (kk % TB == TB - 1))
    def _():
        m = tile_min[...]
        ti = tile_idx[...]

        @pl.when(kk == TB - 1)
        def _():
            run_i[...] = ti
            run_d[...] = m
            run_v[...] = m.astype(jnp.bfloat16).astype(jnp.float32)

        @pl.when(kk > TB - 1)
        def _():
            take = m < run_v[...]
            run_i[...] = jnp.where(take, ti, run_i[...])
            run_d[...] = jnp.where(take, m, run_d[...])
            run_v[...] = jnp.where(take, m, run_v[...]).astype(
                jnp.bfloat16).astype(jnp.float32)

    @pl.when(k == NK)
    def _():
        idx_ref[0] = run_i[...]
        mind_ref[0] = run_d[...]


def _dist_argmin(z_flat, codebook, zsq, csq):
    idx, mind = pl.pallas_call(
        _dist_argmin_kernel,
        grid=(NB, NK + 1),
        in_specs=[
            pl.BlockSpec((BM, D), lambda i, k: (i, 0)),
            pl.BlockSpec((BK, D), lambda i, k: (jnp.minimum(k, NK - 1), 0)),
            pl.BlockSpec((BM, 1), lambda i, k: (i, 0)),
            pl.BlockSpec((1, BK), lambda i, k: (0, jnp.maximum(k - 1, 0))),
        ],
        out_specs=[
            pl.BlockSpec((1, BM, 1), lambda i, k: (i, 0, 0)),
            pl.BlockSpec((1, BM, 1), lambda i, k: (i, 0, 0)),
        ],
        out_shape=[
            jax.ShapeDtypeStruct((NB, BM, 1), jnp.int32),
            jax.ShapeDtypeStruct((NB, BM, 1), jnp.float32),
        ],
        scratch_shapes=[
            pltpu.VMEM((2, BM, BK), jnp.float32),
            pltpu.VMEM((BM, 1), jnp.float32),
            pltpu.VMEM((BM, 1), jnp.int32),
            pltpu.VMEM((BM, 1), jnp.float32),
            pltpu.VMEM((BM, 1), jnp.int32),
            pltpu.VMEM((BM, 1), jnp.float32),
        ],
        compiler_params=pltpu.CompilerParams(
            dimension_semantics=("parallel", "arbitrary"),
            vmem_limit_bytes=67108864),
    )(z_flat, codebook, zsq, csq)
    return idx.reshape(B), mind.reshape(B)


_SC_INFO = plsc.get_sparse_core_info()
_NC = _SC_INFO.num_cores
_NS = _SC_INFO.num_subcores
_NW = _NC * _NS
_BPW = B // _NW
_CH = 8
_NCH = _BPW // _CH


def _gather_sc_kernel(idx_hbm, table_hbm, out_hbm, idx_v, buf0, buf1,
                      sem0, sem1):
    wid = lax.axis_index("s") * _NC + lax.axis_index("c")
    base = wid * _BPW
    pltpu.sync_copy(idx_hbm.at[pl.ds(base, _BPW)], idx_v)
    bufs = (buf0, buf1)
    sems = (sem0, sem1)
    copies = []
    for t in range(_NCH):
        cp = pltpu.async_copy(
            table_hbm.at[idx_v.at[pl.ds(t * _CH, _CH)]], bufs[t % 2],
            sems[t % 2])
        copies.append(cp)
        if t >= 1:
            copies[t - 1].wait()
            pltpu.sync_copy(bufs[(t - 1) % 2],
                            out_hbm.at[pl.ds(base + (t - 1) * _CH, _CH)])
    copies[-1].wait()
    pltpu.sync_copy(bufs[(_NCH - 1) % 2],
                    out_hbm.at[pl.ds(base + (_NCH - 1) * _CH, _CH)])


@functools.partial(
    pl.kernel,
    mesh=plsc.VectorSubcoreMesh(core_axis_name="c", subcore_axis_name="s"),
    out_type=jax.ShapeDtypeStruct((B, D), jnp.float32),
    scratch_types=[
        pltpu.VMEM((_BPW,), jnp.int32),
        pltpu.VMEM((_CH, D), jnp.float32),
        pltpu.VMEM((_CH, D), jnp.float32),
        pltpu.SemaphoreType.DMA,
        pltpu.SemaphoreType.DMA,
    ],
)
def _gather(idx_hbm, table_hbm, out_hbm, idx_v, buf0, buf1, sem0, sem1):
    _gather_sc_kernel(idx_hbm, table_hbm, out_hbm, idx_v, buf0, buf1,
                      sem0, sem1)


def kernel(z, codebook):
    z_flat = z.reshape(B, -1)
    zsq = jnp.sum(z_flat ** 2, axis=1, keepdims=True)
    csq = jnp.sum(codebook ** 2, axis=1).reshape(1, K)
    indices, mind = _dist_argmin(z_flat, codebook, zsq, csq)
    quantized_flat = _gather(indices, codebook)
    loss = 1.25 * (jnp.sum(mind) / (B * D))
    quantized = quantized_flat.reshape(z.shape)
    return (loss, quantized, indices)


# speedup vs baseline: 1.1604x; 1.0762x over previous
import functools

import jax
import jax.numpy as jnp
from jax import lax
from jax.experimental import pallas as pl
from jax.experimental.pallas import tpu as pltpu
from jax.experimental.pallas import tpu_sc as plsc

B = 4096
K = 8192
D = 4096
BM = 1024
BK = 512
TILE = 2048
TB = TILE // BK
NB = B // BM
NK = K // BK


def _dist_argmin_kernel(z_ref, c_ref, zsq_ref, csq_ref, idx_ref, mind_ref,
                        tile_min, tile_idx, run_v, run_i, run_d):
    k = pl.program_id(1)
    zb = z_ref[...]
    cb = c_ref[...]
    dot = lax.dot_general(zb, cb, (((1,), (1,)), ((), ())),
                          preferred_element_type=jnp.float32)
    d = (zsq_ref[...] + csq_ref[...]) - 2.0 * dot

    lmin = jnp.min(d, axis=1, keepdims=True)
    cols = lax.broadcasted_iota(jnp.int32, (1, BK), 1)
    lidx = jnp.min(jnp.where(d == lmin, cols, K), axis=1,
                   keepdims=True) + k * BK

    @pl.when(k % TB == 0)
    def _():
        tile_min[...] = lmin
        tile_idx[...] = lidx

    @pl.when(k % TB != 0)
    def _():
        better = lmin < tile_min[...]
        tile_min[...] = jnp.where(better, lmin, tile_min[...])
        tile_idx[...] = jnp.where(better, lidx, tile_idx[...])

    @pl.when(k % TB == TB - 1)
    def _():
        m = tile_min[...]
        ti = tile_idx[...]

        @pl.when(k == TB - 1)
        def _():
            run_i[...] = ti
            run_d[...] = m
            run_v[...] = m.astype(jnp.bfloat16).astype(jnp.float32)

        @pl.when(k > TB - 1)
        def _():
            take = m < run_v[...]
            run_i[...] = jnp.where(take, ti, run_i[...])
            run_d[...] = jnp.where(take, m, run_d[...])
            run_v[...] = jnp.where(take, m, run_v[...]).astype(
                jnp.bfloat16).astype(jnp.float32)

    @pl.when(k == NK - 1)
    def _():
        idx_ref[0] = run_i[...]
        mind_ref[0] = run_d[...]


def _dist_argmin(z_flat, codebook, zsq, csq):
    idx, mind = pl.pallas_call(
        _dist_argmin_kernel,
        grid=(NB, NK),
        in_specs=[
            pl.BlockSpec((BM, D), lambda i, k: (i, 0)),
            pl.BlockSpec((BK, D), lambda i, k: (k, 0)),
            pl.BlockSpec((BM, 1), lambda i, k: (i, 0)),
            pl.BlockSpec((1, BK), lambda i, k: (0, k)),
        ],
        out_specs=[
            pl.BlockSpec((1, BM, 1), lambda i, k: (i, 0, 0)),
            pl.BlockSpec((1, BM, 1), lambda i, k: (i, 0, 0)),
        ],
        out_shape=[
            jax.ShapeDtypeStruct((NB, BM, 1), jnp.int32),
            jax.ShapeDtypeStruct((NB, BM, 1), jnp.float32),
        ],
        scratch_shapes=[
            pltpu.VMEM((BM, 1), jnp.float32),
            pltpu.VMEM((BM, 1), jnp.int32),
            pltpu.VMEM((BM, 1), jnp.float32),
            pltpu.VMEM((BM, 1), jnp.int32),
            pltpu.VMEM((BM, 1), jnp.float32),
        ],
        compiler_params=pltpu.CompilerParams(
            dimension_semantics=("parallel", "arbitrary"),
            vmem_limit_bytes=67108864),
    )(z_flat, codebook, zsq, csq)
    return idx.reshape(B), mind.reshape(B)


_SC_INFO = plsc.get_sparse_core_info()
_NC = _SC_INFO.num_cores
_NS = _SC_INFO.num_subcores
_NW = _NC * _NS
_BPW = B // _NW
_CH = 8
_NCH = _BPW // _CH


def _gather_sc_kernel(idx_hbm, table_hbm, out_hbm, idx_v, buf0, buf1,
                      sem0, sem1):
    wid = lax.axis_index("s") * _NC + lax.axis_index("c")
    base = wid * _BPW
    pltpu.sync_copy(idx_hbm.at[pl.ds(base, _BPW)], idx_v)
    bufs = (buf0, buf1)
    sems = (sem0, sem1)
    copies = []
    for t in range(_NCH):
        cp = pltpu.async_copy(
            table_hbm.at[idx_v.at[pl.ds(t * _CH, _CH)]], bufs[t % 2],
            sems[t % 2])
        copies.append(cp)
        if t >= 1:
            copies[t - 1].wait()
            pltpu.sync_copy(bufs[(t - 1) % 2],
                            out_hbm.at[pl.ds(base + (t - 1) * _CH, _CH)])
    copies[-1].wait()
    pltpu.sync_copy(bufs[(_NCH - 1) % 2],
                    out_hbm.at[pl.ds(base + (_NCH - 1) * _CH, _CH)])


@functools.partial(
    pl.kernel,
    mesh=plsc.VectorSubcoreMesh(core_axis_name="c", subcore_axis_name="s"),
    out_type=jax.ShapeDtypeStruct((B, D), jnp.float32),
    scratch_types=[
        pltpu.VMEM((_BPW,), jnp.int32),
        pltpu.VMEM((_CH, D), jnp.float32),
        pltpu.VMEM((_CH, D), jnp.float32),
        pltpu.SemaphoreType.DMA,
        pltpu.SemaphoreType.DMA,
    ],
)
def _gather(idx_hbm, table_hbm, out_hbm, idx_v, buf0, buf1, sem0, sem1):
    _gather_sc_kernel(idx_hbm, table_hbm, out_hbm, idx_v, buf0, buf1,
                      sem0, sem1)


def kernel(z, codebook):
    z_flat = z.reshape(B, -1)
    zsq = jnp.sum(z_flat ** 2, axis=1, keepdims=True)
    csq = jnp.sum(codebook ** 2, axis=1).reshape(1, K)
    indices, mind = _dist_argmin(z_flat, codebook, zsq, csq)
    quantized_flat = _gather(indices, codebook)
    loss = 1.25 * (jnp.sum(mind) / (B * D))
    quantized = quantized_flat.reshape(z.shape)
    return (loss, quantized, indices)
